# Initial kernel scaffold; baseline (speedup 1.0000x reference)
import jax
import jax.numpy as jnp
from jax.experimental import pallas as pl


def kernel(dense_input, sparse_input, emb_table, W1, W2, W3, W4):
    raise NotImplementedError("write your pallas kernel here")


# speedup vs baseline: 20.4313x; 20.4313x over previous
import functools

import jax
import jax.numpy as jnp
from jax import lax
from jax.experimental import pallas as pl
from jax.experimental.pallas import tpu as pltpu
from jax.experimental.pallas import tpu_sc as plsc

B = 16384
VOCAB = 4823
EMBED = 64
N_FIELDS = 26
HIDDEN = 256


def _prep_body(emb_ref, w4b_ref, w3_ref, w4a_ref, pt_ref, w34_ref):
    pt_ref[:] = lax.dot_general(
        w4b_ref[:], emb_ref[:], (((1,), (1,)), ((), ())),
        preferred_element_type=jnp.float32)
    w34_ref[:] = lax.dot_general(
        w4a_ref[:], w3_ref[:], (((1,), (1,)), ((), ())),
        preferred_element_type=jnp.float32)


def _mlp_body(x_ref, w1_ref, w2_ref, w34_ref, s_ref, o_ref):
    h = jnp.maximum(
        jnp.dot(x_ref[:], w1_ref[:], preferred_element_type=jnp.float32), 0.0)
    h = jnp.maximum(
        jnp.dot(h, w2_ref[:], preferred_element_type=jnp.float32), 0.0)
    d = jnp.sum(h * w34_ref[:], axis=1, keepdims=True)
    o_ref[:] = jax.nn.sigmoid(d + s_ref[:])


def _make_sc_gather(num_workers, rows_per_w, chunk):
    mesh = plsc.VectorSubcoreMesh(core_axis_name="c", subcore_axis_name="s")
    n_chunks = rows_per_w // chunk
    groups = chunk // 16

    @functools.partial(
        pl.kernel,
        mesh=mesh,
        out_type=jax.ShapeDtypeStruct((B,), jnp.float32),
        compiler_params=pltpu.CompilerParams(needs_layout_passes=False),
        scratch_types=[
            pltpu.VMEM((N_FIELDS * VOCAB,), jnp.float32),
            pltpu.VMEM((chunk * N_FIELDS,), jnp.int32),
            pltpu.VMEM((rows_per_w,), jnp.float32),
        ],
    )
    def sc_gather(tab_hbm, idx_hbm, out_hbm, tab_v, idx_v, out_v):
        nc = 2
        wid = lax.axis_index("s") * nc + lax.axis_index("c")
        base = wid * rows_per_w
        pltpu.sync_copy(tab_hbm, tab_v)
        lane = lax.iota(jnp.int32, 16)
        lane26 = lane * N_FIELDS
        for c in range(n_chunks):
            pltpu.sync_copy(
                idx_hbm.at[pl.ds((base + c * chunk) * N_FIELDS,
                                 chunk * N_FIELDS)],
                idx_v)

            def group_body(g, _, c=c):
                acc = jnp.zeros((16,), jnp.float32)
                for f in range(N_FIELDS):
                    iv = plsc.load_gather(
                        idx_v, [lane26 + (g * (16 * N_FIELDS) + f)])
                    acc = acc + plsc.load_gather(tab_v, [iv + f * VOCAB])
                out_v[pl.ds(c * chunk + g * 16, 16)] = acc
                return 0

            lax.fori_loop(0, groups, group_body, 0)
        pltpu.sync_copy(out_v, out_hbm.at[pl.ds(base, rows_per_w)])

    return sc_gather


def kernel(dense_input, sparse_input, emb_table, W1, W2, W3, W4):
    w4a = W4[:HIDDEN, 0].reshape(1, HIDDEN)
    w4b = W4[HIDDEN:, 0].reshape(N_FIELDS, EMBED)

    pt, w34 = pl.pallas_call(
        _prep_body,
        out_shape=[
            jax.ShapeDtypeStruct((N_FIELDS, VOCAB), jnp.float32),
            jax.ShapeDtypeStruct((1, HIDDEN), jnp.float32),
        ],
    )(emb_table, w4b, W3, w4a)

    info = plsc.get_sparse_core_info()
    num_workers = info.num_cores * info.num_subcores
    rows_per_w = B // num_workers

    sc_gather = _make_sc_gather(num_workers, rows_per_w, chunk=128)
    s = sc_gather(pt.reshape(-1), sparse_input.reshape(-1))

    bm = 1024
    out = pl.pallas_call(
        _mlp_body,
        grid=(B // bm,),
        in_specs=[
            pl.BlockSpec((bm, 13), lambda i: (i, 0)),
            pl.BlockSpec((13, HIDDEN), lambda i: (0, 0)),
            pl.BlockSpec((HIDDEN, HIDDEN), lambda i: (0, 0)),
            pl.BlockSpec((1, HIDDEN), lambda i: (0, 0)),
            pl.BlockSpec((bm, 1), lambda i: (i, 0)),
        ],
        out_specs=pl.BlockSpec((bm, 1), lambda i: (i, 0)),
        out_shape=jax.ShapeDtypeStruct((B, 1), jnp.float32),
    )(dense_input, W1, W2, w34, s.reshape(B, 1))
    return out


# speedup vs baseline: 33.0867x; 1.6194x over previous
import functools

import jax
import jax.numpy as jnp
from jax import lax
from jax.experimental import pallas as pl
from jax.experimental.pallas import tpu as pltpu
from jax.experimental.pallas import tpu_sc as plsc

B = 16384
VOCAB = 4823
EMBED = 64
N_FIELDS = 26
HIDDEN = 256


def _prep_body(embt_ref, w4b_ref, w3_ref, w4a_ref, pt_ref, w34_ref):
    pt_ref[:] = lax.dot_general(
        w4b_ref[:], embt_ref[:], (((1,), (0,)), ((), ())),
        preferred_element_type=jnp.float32)
    w34_ref[:] = lax.dot_general(
        w4a_ref[:], w3_ref[:], (((1,), (1,)), ((), ())),
        preferred_element_type=jnp.float32)


def _mlp_body(xt_ref, w1_ref, w2_ref, w34_ref, d_ref):
    h = jnp.maximum(
        lax.dot_general(xt_ref[:], w1_ref[:], (((0,), (0,)), ((), ())),
                        preferred_element_type=jnp.float32), 0.0)
    h = jnp.maximum(
        jnp.dot(h, w2_ref[:], preferred_element_type=jnp.float32), 0.0)
    d_ref[:] = jnp.sum(h * w34_ref[:], axis=1)


def _combine_body(d_ref, s_ref, o_ref):
    o_ref[:] = jax.nn.sigmoid(d_ref[:] + s_ref[:])


def _make_sc_gather(num_workers, rows_per_w, chunk):
    mesh = plsc.VectorSubcoreMesh(core_axis_name="c", subcore_axis_name="s")
    n_chunks = rows_per_w // chunk
    groups = chunk // 16

    @functools.partial(
        pl.kernel,
        mesh=mesh,
        out_type=jax.ShapeDtypeStruct((B,), jnp.float32),
        compiler_params=pltpu.CompilerParams(
            needs_layout_passes=False, use_tc_tiling_on_sc=False),
        scratch_types=[
            pltpu.VMEM((N_FIELDS, VOCAB), jnp.float32),
            pltpu.VMEM((N_FIELDS, chunk), jnp.int32),
            pltpu.VMEM((rows_per_w,), jnp.float32),
        ],
    )
    def sc_gather(tab_hbm, idx_hbm, out_hbm, tab_v, idx_v, out_v):
        nc = 2
        wid = lax.axis_index("s") * nc + lax.axis_index("c")
        base = wid * rows_per_w
        pltpu.sync_copy(tab_hbm, tab_v)
        for c in range(n_chunks):
            pltpu.sync_copy(idx_hbm.at[:, pl.ds(base + c * chunk, chunk)],
                            idx_v)

            def group_body(g, _, c=c):
                acc = jnp.zeros((16,), jnp.float32)
                for f in range(N_FIELDS):
                    fv = jnp.full((16,), f, jnp.int32)
                    iv = idx_v[f, pl.ds(g * 16, 16)]
                    acc = acc + plsc.load_gather(tab_v, [fv, iv])
                out_v[pl.ds(c * chunk + g * 16, 16)] = acc
                return 0

            lax.fori_loop(0, groups, group_body, 0)
        pltpu.sync_copy(out_v, out_hbm.at[pl.ds(base, rows_per_w)])

    return sc_gather


def kernel(dense_input, sparse_input, emb_table, W1, W2, W3, W4):
    w4a = W4[:HIDDEN, 0].reshape(1, HIDDEN)
    w4b = W4[HIDDEN:, 0].reshape(N_FIELDS, EMBED)

    pt, w34 = pl.pallas_call(
        _prep_body,
        out_shape=[
            jax.ShapeDtypeStruct((N_FIELDS, VOCAB), jnp.float32),
            jax.ShapeDtypeStruct((1, HIDDEN), jnp.float32),
        ],
    )(emb_table.T, w4b, W3, w4a)

    info = plsc.get_sparse_core_info()
    num_workers = info.num_cores * info.num_subcores
    rows_per_w = B // num_workers

    sc_gather = _make_sc_gather(num_workers, rows_per_w, chunk=128)
    s = sc_gather(pt, sparse_input.T)

    bm = 1024
    d = pl.pallas_call(
        _mlp_body,
        grid=(B // bm,),
        in_specs=[
            pl.BlockSpec((13, bm), lambda i: (0, i)),
            pl.BlockSpec((13, HIDDEN), lambda i: (0, 0)),
            pl.BlockSpec((HIDDEN, HIDDEN), lambda i: (0, 0)),
            pl.BlockSpec((1, HIDDEN), lambda i: (0, 0)),
        ],
        out_specs=pl.BlockSpec((bm,), lambda i: (i,)),
        out_shape=jax.ShapeDtypeStruct((B,), jnp.float32),
    )(dense_input.T, W1, W2, w34)

    out = pl.pallas_call(
        _combine_body,
        grid=(1,),
        in_specs=[
            pl.BlockSpec((B,), lambda i: (0,)),
            pl.BlockSpec((B,), lambda i: (0,)),
        ],
        out_specs=pl.BlockSpec((B,), lambda i: (0,)),
        out_shape=jax.ShapeDtypeStruct((B,), jnp.float32),
    )(d, s)
    return out.reshape(B, 1)


# speedup vs baseline: 40.1147x; 1.2124x over previous
import functools

import jax
import jax.numpy as jnp
from jax import lax
from jax.experimental import pallas as pl
from jax.experimental.pallas import tpu as pltpu
from jax.experimental.pallas import tpu_sc as plsc

B = 16384
VOCAB = 4823
EMBED = 64
N_FIELDS = 26
HIDDEN = 256


def _prep_body(embt_ref, w4b_ref, w3_ref, w4a_ref, pt_ref, w34_ref):
    pt_ref[:] = lax.dot_general(
        w4b_ref[:], embt_ref[:], (((1,), (0,)), ((), ())),
        preferred_element_type=jnp.float32)
    w34_ref[:] = lax.dot_general(
        w3_ref[:], w4a_ref[:], (((1,), (0,)), ((), ())),
        preferred_element_type=jnp.float32)


def _mlp_body(xt_ref, w1t_ref, w2t_ref, w34_ref, d_ref):
    h = jnp.maximum(
        jnp.dot(w1t_ref[:], xt_ref[:], preferred_element_type=jnp.float32),
        0.0)
    h = jnp.maximum(
        jnp.dot(w2t_ref[:], h, preferred_element_type=jnp.float32), 0.0)
    d_ref[:] = jnp.sum(h * w34_ref[:], axis=0)


def _combine_body(d_ref, s_ref, o_ref):
    o_ref[:] = jax.nn.sigmoid(d_ref[:] + s_ref[:])


def _make_sc_gather(num_workers, rows_per_w, chunk):
    mesh = plsc.VectorSubcoreMesh(core_axis_name="c", subcore_axis_name="s")
    n_chunks = rows_per_w // chunk
    groups = chunk // 16

    @functools.partial(
        pl.kernel,
        mesh=mesh,
        out_type=jax.ShapeDtypeStruct((B,), jnp.float32),
        compiler_params=pltpu.CompilerParams(
            needs_layout_passes=False, use_tc_tiling_on_sc=False),
        scratch_types=[
            pltpu.VMEM((N_FIELDS, VOCAB), jnp.float32),
            pltpu.VMEM((N_FIELDS, chunk), jnp.int32),
            pltpu.VMEM((rows_per_w,), jnp.float32),
        ],
    )
    def sc_gather(tab_hbm, idx_hbm, out_hbm, tab_v, idx_v, out_v):
        nc = 2
        wid = lax.axis_index("s") * nc + lax.axis_index("c")
        base = wid * rows_per_w
        pltpu.sync_copy(tab_hbm, tab_v)
        for c in range(n_chunks):
            pltpu.sync_copy(idx_hbm.at[:, pl.ds(base + c * chunk, chunk)],
                            idx_v)

            def group_body(g, _, c=c):
                acc = jnp.zeros((16,), jnp.float32)
                for f in range(N_FIELDS):
                    fv = jnp.full((16,), f, jnp.int32)
                    iv = idx_v[f, pl.ds(g * 16, 16)]
                    acc = acc + plsc.load_gather(tab_v, [fv, iv])
                out_v[pl.ds(c * chunk + g * 16, 16)] = acc
                return 0

            lax.fori_loop(0, groups, group_body, 0)
        pltpu.sync_copy(out_v, out_hbm.at[pl.ds(base, rows_per_w)])

    return sc_gather


def kernel(dense_input, sparse_input, emb_table, W1, W2, W3, W4):
    w4a = W4[:HIDDEN]
    w4b = W4[HIDDEN:, 0].reshape(N_FIELDS, EMBED)

    pt, w34 = pl.pallas_call(
        _prep_body,
        out_shape=[
            jax.ShapeDtypeStruct((N_FIELDS, VOCAB), jnp.float32),
            jax.ShapeDtypeStruct((HIDDEN, 1), jnp.float32),
        ],
    )(emb_table.T, w4b, W3, w4a)

    info = plsc.get_sparse_core_info()
    num_workers = info.num_cores * info.num_subcores
    rows_per_w = B // num_workers

    sc_gather = _make_sc_gather(num_workers, rows_per_w, chunk=128)
    s = sc_gather(pt, sparse_input.T)

    bm = 1024
    d = pl.pallas_call(
        _mlp_body,
        grid=(B // bm,),
        in_specs=[
            pl.BlockSpec((13, bm), lambda i: (0, i)),
            pl.BlockSpec((HIDDEN, 13), lambda i: (0, 0)),
            pl.BlockSpec((HIDDEN, HIDDEN), lambda i: (0, 0)),
            pl.BlockSpec((HIDDEN, 1), lambda i: (0, 0)),
        ],
        out_specs=pl.BlockSpec((bm,), lambda i: (i,)),
        out_shape=jax.ShapeDtypeStruct((B,), jnp.float32),
    )(dense_input.T, W1.T, W2.T, w34)

    out = pl.pallas_call(
        _combine_body,
        grid=(1,),
        in_specs=[
            pl.BlockSpec((B,), lambda i: (0,)),
            pl.BlockSpec((B,), lambda i: (0,)),
        ],
        out_specs=pl.BlockSpec((B,), lambda i: (0,)),
        out_shape=jax.ShapeDtypeStruct((B,), jnp.float32),
    )(d, s)
    return out.reshape(B, 1)
